# baseline (speedup 1.0000x reference)
import functools

import jax
import jax.numpy as jnp
from jax import lax
from jax.experimental import pallas as pl
from jax.experimental.pallas import tpu as pltpu
from jax.experimental.pallas import tpu_sc as plsc

NC = 2
NS = 16
NW = NC * NS
B = 128
LANES = 16


def _sc_mesh():
  return plsc.VectorSubcoreMesh(core_axis_name="c", subcore_axis_name="s")


_SC_PARAMS = pltpu.CompilerParams(use_tc_tiling_on_sc=False)


def _deg_hist(cols3, npad, chunks):
  rows_per_sub = npad // NS

  @functools.partial(
      pl.kernel,
      out_type=jax.ShapeDtypeStruct((NC, npad, LANES), jnp.float32),
      mesh=_sc_mesh(),
      scratch_types=[
          pltpu.VMEM((chunks + 4, B), jnp.int32),
          pltpu.VMEM((B, LANES), jnp.float32),
          pltpu.VMEM_SHARED((npad, LANES), jnp.float32),
      ],
      compiler_params=_SC_PARAMS,
  )
  def k(col_hbm, out_hbm, col_v, ones_v, acc):
    c = lax.axis_index("c")
    s = lax.axis_index("s")
    wid = s * NC + c
    pltpu.sync_copy(col_hbm.at[wid], col_v)

    @pl.loop(0, B)
    def _(i):
      ones_v[i, pl.ds(0, LANES)] = jnp.zeros((LANES,), jnp.float32)

    @pl.loop(0, rows_per_sub, step=B)
    def _(r):
      pltpu.sync_copy(ones_v, acc.at[pl.ds(s * rows_per_sub + r, B)])

    @pl.loop(0, B)
    def _(i):
      ones_v[i, pl.ds(0, LANES)] = jnp.ones((LANES,), jnp.float32)

    plsc.subcore_barrier()

    @pl.loop(0, chunks)
    def _(j):
      pltpu.sync_copy(ones_v, acc.at[col_v.at[j]], add=True)

    plsc.subcore_barrier()
    sl = pl.ds(s * rows_per_sub, rows_per_sub)
    pltpu.sync_copy(acc.at[sl], out_hbm.at[c].at[sl])

  return k(cols3)


def _edge_agg(g, rows3, cols3, npad, feat, chunks):
  rows_per_sub = npad // NS

  half = chunks // 2
  nbuf = 2 if feat > 64 else 4

  @functools.partial(
      pl.kernel,
      out_type=jax.ShapeDtypeStruct((NC, npad, feat), jnp.float32),
      mesh=_sc_mesh(),
      scratch_types=[
          pltpu.VMEM((half + nbuf, B), jnp.int32),
          pltpu.VMEM((half + nbuf, B), jnp.int32),
          pltpu.VMEM((nbuf, B, feat), jnp.float32),
          pltpu.VMEM_SHARED((npad, feat), jnp.float32),
      ] + [pltpu.SemaphoreType.DMA] * nbuf,
      compiler_params=_SC_PARAMS,
  )
  def k(g_hbm, row_hbm, col_hbm, out_hbm, row_v, col_v, buf, acc, *sems):
    c = lax.axis_index("c")
    s = lax.axis_index("s")
    wid = s * NC + c

    @pl.loop(0, B)
    def _(i):
      for j in range(feat // LANES):
        buf[0, i, pl.ds(j * LANES, LANES)] = jnp.zeros((LANES,), jnp.float32)

    @pl.loop(0, rows_per_sub, step=B)
    def _(r):
      pltpu.sync_copy(buf.at[0], acc.at[pl.ds(s * rows_per_sub + r, B)])

    plsc.subcore_barrier()

    for phase in range(2):
      pltpu.sync_copy(row_hbm.at[wid].at[pl.ds(phase * half, half + nbuf)],
                      row_v)
      pltpu.sync_copy(col_hbm.at[wid].at[pl.ds(phase * half, half + nbuf)],
                      col_v)
      for b in range(nbuf):
        pltpu.async_copy(g_hbm.at[row_v.at[b]], buf.at[b], sems[b])

      @pl.loop(0, half, step=nbuf)
      def _(j):
        for b in range(nbuf):
          pltpu.make_async_copy(g_hbm.at[row_v.at[j + b]], buf.at[b],
                                sems[b]).wait()
          pltpu.sync_copy(buf.at[b], acc.at[col_v.at[j + b]], add=True)
          pltpu.async_copy(g_hbm.at[row_v.at[j + nbuf + b]], buf.at[b],
                           sems[b])

      for b in range(nbuf):
        pltpu.make_async_copy(g_hbm.at[row_v.at[half + b]], buf.at[b],
                              sems[b]).wait()

    plsc.subcore_barrier()
    sl = pl.ds(s * rows_per_sub, rows_per_sub)
    pltpu.sync_copy(acc.at[sl], out_hbm.at[c].at[sl])

  return k(g, rows3, cols3)


def _tc_matmul(xp, w):
  m, kdim = xp.shape
  f = w.shape[1]
  bm = 1024

  def body(x_ref, w_ref, o_ref):
    o_ref[...] = jnp.dot(x_ref[...], w_ref[...],
                         preferred_element_type=jnp.float32)

  return pl.pallas_call(
      body,
      grid=(m // bm,),
      in_specs=[pl.BlockSpec((bm, kdim), lambda i: (i, 0)),
                pl.BlockSpec((kdim, f), lambda i: (0, 0))],
      out_specs=pl.BlockSpec((bm, f), lambda i: (i, 0)),
      out_shape=jax.ShapeDtypeStruct((m, f), jnp.float32),
  )(xp, w)


def _scale(h1, da, db, n_real):
  m, h = h1.shape
  bm = 1024

  def body(h_ref, da_ref, db_ref, dinv_ref, g_ref):
    i = pl.program_id(0)
    rowid = lax.broadcasted_iota(jnp.int32, (bm, 1), 0) + i * bm
    deg = (da_ref[:, 0:1] + db_ref[:, 0:1]
           + jnp.where(rowid < n_real, 1.0, 0.0))
    dinv = jnp.where(deg > 0, lax.rsqrt(jnp.maximum(deg, 1e-12)), 0.0)
    dinv_ref[...] = jnp.broadcast_to(dinv, (bm, LANES))
    g_ref[...] = dinv * h_ref[...]

  return pl.pallas_call(
      body,
      grid=(m // bm,),
      in_specs=[pl.BlockSpec((bm, h), lambda i: (i, 0)),
                pl.BlockSpec((bm, LANES), lambda i: (i, 0)),
                pl.BlockSpec((bm, LANES), lambda i: (i, 0))],
      out_specs=[pl.BlockSpec((bm, LANES), lambda i: (i, 0)),
                 pl.BlockSpec((bm, h), lambda i: (i, 0))],
      out_shape=[jax.ShapeDtypeStruct((m, LANES), jnp.float32),
                 jax.ShapeDtypeStruct((m, h), jnp.float32)],
  )(h1, da, db)


def _layer2_in(s1a, s1b, g1, dinv, b1r, w2):
  m, h = g1.shape
  c = w2.shape[1]
  bm = 1024

  def body(sa, sb, g, dv, b, w, g2_ref):
    d = dv[:, 0:1]
    z = jnp.maximum(d * (sa[...] + sb[...] + g[...]) + b[0:1, :], 0.0)
    g2_ref[...] = d * jnp.dot(z, w[...], preferred_element_type=jnp.float32)

  return pl.pallas_call(
      body,
      grid=(m // bm,),
      in_specs=[pl.BlockSpec((bm, h), lambda i: (i, 0)),
                pl.BlockSpec((bm, h), lambda i: (i, 0)),
                pl.BlockSpec((bm, h), lambda i: (i, 0)),
                pl.BlockSpec((bm, LANES), lambda i: (i, 0)),
                pl.BlockSpec((1, h), lambda i: (0, 0)),
                pl.BlockSpec((h, c), lambda i: (0, 0))],
      out_specs=pl.BlockSpec((bm, c), lambda i: (i, 0)),
      out_shape=jax.ShapeDtypeStruct((m, c), jnp.float32),
  )(s1a, s1b, g1, dinv, b1r, w2)


def _finish(s2a, s2b, g2, dinv, b2r):
  m, c = g2.shape
  bm = 1024

  def body(sa, sb, g, dv, b, o_ref):
    d = dv[:, 0:1]
    o_ref[...] = d * (sa[...] + sb[...] + g[...]) + b[0:1, :]

  return pl.pallas_call(
      body,
      grid=(m // bm,),
      in_specs=[pl.BlockSpec((bm, c), lambda i: (i, 0)),
                pl.BlockSpec((bm, c), lambda i: (i, 0)),
                pl.BlockSpec((bm, c), lambda i: (i, 0)),
                pl.BlockSpec((bm, LANES), lambda i: (i, 0)),
                pl.BlockSpec((1, c), lambda i: (0, 0))],
      out_specs=pl.BlockSpec((bm, c), lambda i: (i, 0)),
      out_shape=jax.ShapeDtypeStruct((m, c), jnp.float32),
  )(s2a, s2b, g2, dinv, b2r)


def _ceil_to(v, mult):
  return (v + mult - 1) // mult * mult


@jax.jit
def kernel(x, adjs, W1, b1, W2, b2):
  n, _ = x.shape
  h = W1.shape[1]
  c = W2.shape[1]
  e = adjs.shape[1]

  npad = _ceil_to(n + 1, NS * B)
  chunks = _ceil_to(_ceil_to(e, NW * B) // (NW * B), 8)
  epad = chunks * NW * B

  row = adjs[0].astype(jnp.int32)
  col = adjs[1].astype(jnp.int32)
  pad_idx = jnp.full((epad - e,), n, jnp.int32)
  extra = jnp.full((NW, 4, B), n, jnp.int32)
  rows3 = jnp.concatenate(
      [jnp.concatenate([row, pad_idx]).reshape(NW, chunks, B), extra], axis=1)
  cols3 = jnp.concatenate(
      [jnp.concatenate([col, pad_idx]).reshape(NW, chunks, B), extra], axis=1)
  xp = jnp.pad(x, ((0, npad - n), (0, 0)))

  degp = _deg_hist(cols3, npad, chunks)
  h1 = _tc_matmul(xp, W1)
  dinv, g1 = _scale(h1, degp[0], degp[1], n)
  s1 = _edge_agg(g1, rows3, cols3, npad, h, chunks)
  g2 = _layer2_in(s1[0], s1[1], g1, dinv, b1.reshape(1, h), W2)
  s2 = _edge_agg(g2, rows3, cols3, npad, c, chunks)
  outp = _finish(s2[0], s2[1], g2, dinv, b2.reshape(1, c))
  return outp[:n]


# speedup vs baseline: 3.1893x; 3.1893x over previous
import functools

import jax
import jax.numpy as jnp
from jax import lax
from jax.experimental import pallas as pl
from jax.experimental.pallas import tpu as pltpu
from jax.experimental.pallas import tpu_sc as plsc

NC = 2
NS = 16
NW = NC * NS
B = 128
LANES = 16
FH = 64


def _sc_mesh():
  return plsc.VectorSubcoreMesh(core_axis_name="c", subcore_axis_name="s")


_SC_PARAMS = pltpu.CompilerParams(use_tc_tiling_on_sc=False)


def _deg_hist(cols3, npad, chunks):
  rows_per_sub = npad // NS

  @functools.partial(
      pl.kernel,
      out_type=jax.ShapeDtypeStruct((NC, npad, LANES), jnp.float32),
      mesh=_sc_mesh(),
      scratch_types=[
          pltpu.VMEM((chunks, B), jnp.int32),
          pltpu.VMEM((B, LANES), jnp.float32),
          pltpu.VMEM_SHARED((npad, LANES), jnp.float32),
      ],
      compiler_params=_SC_PARAMS,
  )
  def k(col_hbm, out_hbm, col_v, ones_v, acc):
    c = lax.axis_index("c")
    s = lax.axis_index("s")
    wid = s * NC + c
    pltpu.sync_copy(col_hbm.at[wid], col_v)

    @pl.loop(0, B)
    def _(i):
      ones_v[i, pl.ds(0, LANES)] = jnp.zeros((LANES,), jnp.float32)

    @pl.loop(0, rows_per_sub, step=B)
    def _(r):
      pltpu.sync_copy(ones_v, acc.at[pl.ds(s * rows_per_sub + r, B)])

    @pl.loop(0, B)
    def _(i):
      ones_v[i, pl.ds(0, LANES)] = jnp.ones((LANES,), jnp.float32)

    plsc.subcore_barrier()

    @pl.loop(0, chunks)
    def _(j):
      pltpu.sync_copy(ones_v, acc.at[col_v.at[j]], add=True)

    plsc.subcore_barrier()
    sl = pl.ds(s * rows_per_sub, rows_per_sub)
    pltpu.sync_copy(acc.at[sl], out_hbm.at[c].at[sl])

  return k(cols3)


def _edge_agg(gh, rows3, cols3, npad, passes, chunks):
  rows_per_sub = npad // NS

  @functools.partial(
      pl.kernel,
      out_type=jax.ShapeDtypeStruct((NC, passes, npad, FH), jnp.float32),
      mesh=_sc_mesh(),
      scratch_types=[
          pltpu.VMEM((chunks, B), jnp.int32),
          pltpu.VMEM((chunks, B), jnp.int32),
          pltpu.VMEM((B, FH), jnp.float32),
          pltpu.VMEM((B, FH), jnp.float32),
          pltpu.VMEM_SHARED((npad, FH), jnp.float32),
          pltpu.VMEM_SHARED((npad, FH), jnp.float32),
          pltpu.SemaphoreType.DMA,
      ],
      compiler_params=_SC_PARAMS,
  )
  def k(g_hbm, row_hbm, col_hbm, out_hbm, row_v, col_v, buf, zbuf, table,
        acc, sem):
    c = lax.axis_index("c")
    s = lax.axis_index("s")
    wid = s * NC + c
    sl = pl.ds(s * rows_per_sub, rows_per_sub)

    pltpu.sync_copy(row_hbm.at[wid], row_v)
    pltpu.sync_copy(col_hbm.at[wid], col_v)

    @pl.loop(0, B)
    def _(i):
      for j in range(FH // LANES):
        zbuf[i, pl.ds(j * LANES, LANES)] = jnp.zeros((LANES,), jnp.float32)

    for p in range(passes):
      pltpu.sync_copy(g_hbm.at[p].at[sl], table.at[sl])

      @pl.loop(0, rows_per_sub, step=B)
      def _(r):
        pltpu.sync_copy(zbuf, acc.at[pl.ds(s * rows_per_sub + r, B)])

      plsc.subcore_barrier()

      @pl.loop(0, chunks)
      def _(j):
        pltpu.async_copy(table.at[row_v.at[j]], buf, sem).wait()
        pltpu.sync_copy(buf, acc.at[col_v.at[j]], add=True)

      plsc.subcore_barrier()
      pltpu.sync_copy(acc.at[sl], out_hbm.at[c].at[p].at[sl])
      plsc.subcore_barrier()

  return k(gh, rows3, cols3)


def _tc_matmul(xp, w):
  m, kdim = xp.shape
  f = w.shape[1]
  bm = 1024

  def body(x_ref, w_ref, o_ref):
    o_ref[...] = jnp.dot(x_ref[...], w_ref[...],
                         preferred_element_type=jnp.float32)

  return pl.pallas_call(
      body,
      grid=(m // bm,),
      in_specs=[pl.BlockSpec((bm, kdim), lambda i: (i, 0)),
                pl.BlockSpec((kdim, f), lambda i: (0, 0))],
      out_specs=pl.BlockSpec((bm, f), lambda i: (i, 0)),
      out_shape=jax.ShapeDtypeStruct((m, f), jnp.float32),
  )(xp, w)


def _scale(h1, da, db, n_real):
  m, h = h1.shape
  bm = 1024
  halves = h // FH

  def body(h_ref, da_ref, db_ref, dinv_ref, g_ref):
    i = pl.program_id(0)
    rowid = lax.broadcasted_iota(jnp.int32, (bm, 1), 0) + i * bm
    deg = (da_ref[:, 0:1] + db_ref[:, 0:1]
           + jnp.where(rowid < n_real, 1.0, 0.0))
    dinv = jnp.where(deg > 0, lax.rsqrt(jnp.maximum(deg, 1e-12)), 0.0)
    dinv_ref[...] = jnp.broadcast_to(dinv, (bm, LANES))
    g = dinv * h_ref[...]
    for p in range(halves):
      g_ref[p, :, :] = g[:, p * FH:(p + 1) * FH]

  return pl.pallas_call(
      body,
      grid=(m // bm,),
      in_specs=[pl.BlockSpec((bm, h), lambda i: (i, 0)),
                pl.BlockSpec((bm, LANES), lambda i: (i, 0)),
                pl.BlockSpec((bm, LANES), lambda i: (i, 0))],
      out_specs=[pl.BlockSpec((bm, LANES), lambda i: (i, 0)),
                 pl.BlockSpec((halves, bm, FH), lambda i: (0, i, 0))],
      out_shape=[jax.ShapeDtypeStruct((m, LANES), jnp.float32),
                 jax.ShapeDtypeStruct((halves, m, FH), jnp.float32)],
  )(h1, da, db)


def _layer2_in(s1, g1h, dinv, b1r, w2):
  nc, halves, m, fh = s1.shape
  h = halves * fh
  c = w2.shape[1]
  bm = 1024

  def body(s_ref, g_ref, dv, b, w, g2_ref):
    d = dv[:, 0:1]
    zs = []
    for p in range(halves):
      t = s_ref[0, p, :, :] + s_ref[1, p, :, :] + g_ref[p, :, :]
      zs.append(jnp.maximum(d * t + b[0:1, p * fh:(p + 1) * fh], 0.0))
    z = jnp.concatenate(zs, axis=1)
    g2_ref[0, :, :] = d * jnp.dot(z, w[...],
                                  preferred_element_type=jnp.float32)

  return pl.pallas_call(
      body,
      grid=(m // bm,),
      in_specs=[pl.BlockSpec((nc, halves, bm, fh), lambda i: (0, 0, i, 0)),
                pl.BlockSpec((halves, bm, fh), lambda i: (0, i, 0)),
                pl.BlockSpec((bm, LANES), lambda i: (i, 0)),
                pl.BlockSpec((1, h), lambda i: (0, 0)),
                pl.BlockSpec((h, c), lambda i: (0, 0))],
      out_specs=pl.BlockSpec((1, bm, c), lambda i: (0, i, 0)),
      out_shape=jax.ShapeDtypeStruct((1, m, c), jnp.float32),
  )(s1, g1h, dinv, b1r, w2)


def _finish(s2, g2h, dinv, b2r):
  nc, _, m, c = s2.shape
  bm = 1024

  def body(s_ref, g_ref, dv, b, o_ref):
    d = dv[:, 0:1]
    o_ref[...] = (d * (s_ref[0, 0, :, :] + s_ref[1, 0, :, :]
                       + g_ref[0, :, :]) + b[0:1, :])

  return pl.pallas_call(
      body,
      grid=(m // bm,),
      in_specs=[pl.BlockSpec((nc, 1, bm, c), lambda i: (0, 0, i, 0)),
                pl.BlockSpec((1, bm, c), lambda i: (0, i, 0)),
                pl.BlockSpec((bm, LANES), lambda i: (i, 0)),
                pl.BlockSpec((1, c), lambda i: (0, 0))],
      out_specs=pl.BlockSpec((bm, c), lambda i: (i, 0)),
      out_shape=jax.ShapeDtypeStruct((m, c), jnp.float32),
  )(s2, g2h, dinv, b2r)


def _ceil_to(v, mult):
  return (v + mult - 1) // mult * mult


@jax.jit
def kernel(x, adjs, W1, b1, W2, b2):
  n, _ = x.shape
  h = W1.shape[1]
  c = W2.shape[1]
  e = adjs.shape[1]

  npad = _ceil_to(n + 1, NS * B)
  chunks = _ceil_to(e, NW * B) // (NW * B)
  epad = chunks * NW * B

  row = adjs[0].astype(jnp.int32)
  col = adjs[1].astype(jnp.int32)
  pad_idx = jnp.full((epad - e,), n, jnp.int32)
  rows3 = jnp.concatenate([row, pad_idx]).reshape(NW, chunks, B)
  cols3 = jnp.concatenate([col, pad_idx]).reshape(NW, chunks, B)
  xp = jnp.pad(x, ((0, npad - n), (0, 0)))

  degp = _deg_hist(cols3, npad, chunks)
  h1 = _tc_matmul(xp, W1)
  dinv, g1h = _scale(h1, degp[0], degp[1], n)
  s1 = _edge_agg(g1h, rows3, cols3, npad, h // FH, chunks)
  g2h = _layer2_in(s1, g1h, dinv, b1.reshape(1, h), W2)
  s2 = _edge_agg(g2h, rows3, cols3, npad, c // FH, chunks)
  outp = _finish(s2, g2h, dinv, b2.reshape(1, c))
  return outp[:n]
